# baseline (speedup 1.0000x reference)
import functools

import jax
import jax.numpy as jnp
from jax import lax
from jax.experimental import pallas as pl
from jax.experimental.pallas import tpu as pltpu
from jax.experimental.pallas import tpu_sc as plsc

B = 4096
S = 200
D = 32
N = B * S

NUM_CORES = 2
NUM_SUBCORES = 16
NW = NUM_CORES * NUM_SUBCORES
NHG = 1600 // NW
HG_ROWS = 512


def _splat(v):
    return jnp.full((16,), v, jnp.int32)


_TBLK = 4096


def _depad_body(x_ref, y_ref):
    xt = lax.dot_general(
        x_ref[...], jnp.eye(32, dtype=jnp.float32),
        (((0,), (0,)), ((), ())),
        preferred_element_type=jnp.float32,
    )
    a = xt.reshape(_TBLK // 4, 4, 32)
    y_ref[...] = jnp.concatenate([a[:, 0], a[:, 1], a[:, 2], a[:, 3]], axis=1)


_depad = pl.pallas_call(
    _depad_body,
    grid=(245,),
    in_specs=[pl.BlockSpec((32, _TBLK), lambda i: (0, i))],
    out_specs=pl.BlockSpec((_TBLK // 4, 128), lambda i: (i, 0)),
    out_shape=jax.ShapeDtypeStruct((250000, 128), jnp.float32),
)


def _body(x4, tok, p4, o3, idx0, idx1, rows0, rows1, t0, t1, pos4_v,
          gsem0, gsem1, ssem0, ssem1):
    wid = lax.axis_index("c") * NUM_SUBCORES + lax.axis_index("s")
    hbase = wid * NHG
    idx = (idx0, idx1)
    rows = (rows0, rows1)
    t2 = (t0, t1)
    gsem = (gsem0, gsem1)
    ssem = (ssem0, ssem1)

    pltpu.sync_copy(p4, pos4_v)

    iota = jnp.arange(16, dtype=jnp.int32)
    a8 = iota >> 3
    r8 = iota & 7

    def coords(h):
        a = h // 64
        bb = (h // 2) % 32
        half = h % 2
        return a, bb, half

    def fire_gathers(h, b):
        a, bb, half = coords(h)
        r0 = pl.multiple_of(half * 4, 4)
        pltpu.sync_copy(x4.at[a, bb, pl.ds(r0, 4), :], idx[b])
        for r in range(4):
            pltpu.async_copy(
                tok.at[idx[b].at[r]], rows[b].at[pl.ds(r * 128, 128), :],
                gsem[b],
            )

    def drain_gathers(b):
        pltpu.make_async_copy(
            tok.at[pl.ds(0, HG_ROWS), :], rows[b], gsem[b]
        ).wait()

    def drain_out(b):
        pltpu.make_async_copy(
            t2[b].at[:, :, pl.ds(0, 128)], o3.at[pl.ds(0, 16), 0, :, :], ssem[b]
        ).wait()

    def transpose_add(h, b):
        a, _, half = coords(h)
        rv = rows[b]
        tv = t2[b]
        for rg in range(4):
            s = a * 8 + half * 4 + rg
            sc_hi = _splat(s // 128)
            sc_lo = _splat(s % 128)
            pos_lo = plsc.load_gather(pos4_v, [a8, sc_hi, r8, sc_lo])
            pos_hi = plsc.load_gather(pos4_v, [a8 + 2, sc_hi, r8, sc_lo])
            iv0 = a8 + (rg * 4)
            iv1 = iv0 + 2

            @plsc.parallel_loop(0, 128, unroll=8)
            def _col(c):
                row = rg * 128 + c
                cv = _splat(c)
                plsc.store_scatter(tv, [iv0, r8, cv], rv[row, pl.ds(0, 16)] + pos_lo)
                plsc.store_scatter(tv, [iv1, r8, cv], rv[row, pl.ds(16, 16)] + pos_hi)

    fire_gathers(hbase, 0)
    fire_gathers(hbase + 1, 1)

    @pl.loop(0, NHG, step=2)
    def _hg(g):
        for b in range(2):
            h = hbase + g + b
            drain_gathers(b)

            @pl.when(g + b >= 2)
            def _():
                drain_out(b)

            transpose_add(h, b)

            a, bb, half = coords(h)
            orow0 = pl.multiple_of(a * 32 + half * 16, 16)
            pltpu.async_copy(
                t2[b].at[:, :, pl.ds(0, 128)],
                o3.at[pl.ds(orow0, 16), bb, :, :], ssem[b],
            )

            @pl.when(g + b + 2 < NHG)
            def _():
                fire_gathers(h + 2, b)

    drain_out(0)
    drain_out(1)


@functools.partial(
    pl.kernel,
    out_type=jax.ShapeDtypeStruct((800, 32, 8, 128), jnp.float32),
    mesh=plsc.VectorSubcoreMesh(core_axis_name="c", subcore_axis_name="s"),
    scratch_types=[
        pltpu.VMEM((4, 128), jnp.int32),
        pltpu.VMEM((4, 128), jnp.int32),
        pltpu.VMEM((HG_ROWS, D), jnp.float32),
        pltpu.VMEM((HG_ROWS, D), jnp.float32),
        pltpu.VMEM((16, 8, 129), jnp.float32),
        pltpu.VMEM((16, 8, 129), jnp.float32),
        pltpu.VMEM((4, 4, 8, 128), jnp.float32),
        pltpu.SemaphoreType.DMA,
        pltpu.SemaphoreType.DMA,
        pltpu.SemaphoreType.DMA,
        pltpu.SemaphoreType.DMA,
    ],
    compiler_params=pltpu.CompilerParams(use_tc_tiling_on_sc=False, needs_layout_passes=False),
)
def _embed(x4, tok, p4, o3, idx0, idx1, rows0, rows1, t0, t1, pos4_v,
           gsem0, gsem1, ssem0, ssem1):
    _body(x4, tok, p4, o3, idx0, idx1, rows0, rows1, t0, t1, pos4_v,
          gsem0, gsem1, ssem0, ssem1)


def kernel(x, token_table, position_table):
    x4 = x.astype(jnp.int32).T.reshape(25, 8, 32, 128).transpose(0, 2, 1, 3)
    p4 = position_table.T.reshape(4, 8, 4, 128).transpose(0, 2, 1, 3)
    tok_lin = _depad(token_table.T).reshape(1000000, 32)
    o3 = _embed(x4, tok_lin, p4)
    out = o3.reshape(S, 4, 32, 8, 128).transpose(2, 4, 0, 1, 3)
    return out.reshape(B, S, D)


# speedup vs baseline: 1.0678x; 1.0678x over previous
import functools

import jax
import jax.numpy as jnp
from jax import lax
from jax.experimental import pallas as pl
from jax.experimental.pallas import tpu as pltpu
from jax.experimental.pallas import tpu_sc as plsc

B = 4096
S = 200
D = 32
N = B * S

NUM_CORES = 2
NUM_SUBCORES = 16
NW = NUM_CORES * NUM_SUBCORES
NHG = 1600 // NW
HG_ROWS = 512


def _splat(v):
    return jnp.full((16,), v, jnp.int32)


_TBLK = 4096


def _depad_body(x_ref, y_ref):
    xt = x_ref[...].T
    a = xt.reshape(_TBLK // 4, 4, 32)
    y_ref[...] = jnp.concatenate([a[:, 0], a[:, 1], a[:, 2], a[:, 3]], axis=1)


_depad = pl.pallas_call(
    _depad_body,
    grid=(245,),
    in_specs=[pl.BlockSpec((32, _TBLK), lambda i: (0, i))],
    out_specs=pl.BlockSpec((_TBLK // 4, 128), lambda i: (i, 0)),
    out_shape=jax.ShapeDtypeStruct((250000, 128), jnp.float32),
    compiler_params=pltpu.CompilerParams(dimension_semantics=("parallel",)),
)


def _body(x4, tok, p4, o3, idx0, idx1, rows0, rows1, t0, t1, pos4_v,
          gsem0, gsem1, ssem0, ssem1):
    wid = lax.axis_index("c") * NUM_SUBCORES + lax.axis_index("s")
    hbase = wid * NHG
    idx = (idx0, idx1)
    rows = (rows0, rows1)
    t2 = (t0, t1)
    gsem = (gsem0, gsem1)
    ssem = (ssem0, ssem1)

    pltpu.sync_copy(p4, pos4_v)

    iota = jnp.arange(16, dtype=jnp.int32)
    a8 = iota >> 3
    r8 = iota & 7

    def coords(h):
        a = h // 64
        bb = (h // 2) % 32
        half = h % 2
        return a, bb, half

    def fire_gathers(h, b):
        a, bb, half = coords(h)
        r0 = pl.multiple_of(half * 4, 4)
        pltpu.sync_copy(x4.at[a, bb, pl.ds(r0, 4), :], idx[b])
        for r in range(4):
            pltpu.async_copy(
                tok.at[idx[b].at[r]], rows[b].at[pl.ds(r * 128, 128), :],
                gsem[b],
            )

    def drain_gathers(b):
        pltpu.make_async_copy(
            tok.at[pl.ds(0, HG_ROWS), :], rows[b], gsem[b]
        ).wait()

    def drain_out(b):
        pltpu.make_async_copy(
            t2[b].at[:, :, pl.ds(0, 128)], o3.at[pl.ds(0, 16), 0, :, :], ssem[b]
        ).wait()

    def transpose_add(h, b):
        a, _, half = coords(h)
        rv = rows[b]
        tv = t2[b]
        for rg in range(4):
            s = a * 8 + half * 4 + rg
            sc_hi = _splat(s // 128)
            sc_lo = _splat(s % 128)
            pos_lo = plsc.load_gather(pos4_v, [a8, sc_hi, r8, sc_lo])
            pos_hi = plsc.load_gather(pos4_v, [a8 + 2, sc_hi, r8, sc_lo])
            iv0 = a8 + (rg * 4)
            iv1 = iv0 + 2

            @plsc.parallel_loop(0, 128, unroll=8)
            def _col(c):
                row = rg * 128 + c
                cv = _splat(c)
                plsc.store_scatter(tv, [iv0, r8, cv], rv[row, pl.ds(0, 16)] + pos_lo)
                plsc.store_scatter(tv, [iv1, r8, cv], rv[row, pl.ds(16, 16)] + pos_hi)

    fire_gathers(hbase, 0)
    fire_gathers(hbase + 1, 1)

    @pl.loop(0, NHG, step=2)
    def _hg(g):
        for b in range(2):
            h = hbase + g + b
            drain_gathers(b)

            @pl.when(g + b >= 2)
            def _():
                drain_out(b)

            transpose_add(h, b)

            a, bb, half = coords(h)
            orow0 = pl.multiple_of(a * 32 + half * 16, 16)
            pltpu.async_copy(
                t2[b].at[:, :, pl.ds(0, 128)],
                o3.at[pl.ds(orow0, 16), bb, :, :], ssem[b],
            )

            @pl.when(g + b + 2 < NHG)
            def _():
                fire_gathers(h + 2, b)

    drain_out(0)
    drain_out(1)


@functools.partial(
    pl.kernel,
    out_type=jax.ShapeDtypeStruct((800, 32, 8, 128), jnp.float32),
    mesh=plsc.VectorSubcoreMesh(core_axis_name="c", subcore_axis_name="s"),
    scratch_types=[
        pltpu.VMEM((4, 128), jnp.int32),
        pltpu.VMEM((4, 128), jnp.int32),
        pltpu.VMEM((HG_ROWS, D), jnp.float32),
        pltpu.VMEM((HG_ROWS, D), jnp.float32),
        pltpu.VMEM((16, 8, 129), jnp.float32),
        pltpu.VMEM((16, 8, 129), jnp.float32),
        pltpu.VMEM((4, 4, 8, 128), jnp.float32),
        pltpu.SemaphoreType.DMA,
        pltpu.SemaphoreType.DMA,
        pltpu.SemaphoreType.DMA,
        pltpu.SemaphoreType.DMA,
    ],
    compiler_params=pltpu.CompilerParams(use_tc_tiling_on_sc=False, needs_layout_passes=False),
)
def _embed(x4, tok, p4, o3, idx0, idx1, rows0, rows1, t0, t1, pos4_v,
           gsem0, gsem1, ssem0, ssem1):
    _body(x4, tok, p4, o3, idx0, idx1, rows0, rows1, t0, t1, pos4_v,
          gsem0, gsem1, ssem0, ssem1)


def kernel(x, token_table, position_table):
    x4 = x.astype(jnp.int32).T.reshape(25, 8, 32, 128).transpose(0, 2, 1, 3)
    p4 = position_table.T.reshape(4, 8, 4, 128).transpose(0, 2, 1, 3)
    tok_lin = _depad(token_table.T).reshape(1000000, 32)
    o3 = _embed(x4, tok_lin, p4)
    out = o3.reshape(S, 4, 32, 8, 128).transpose(2, 4, 0, 1, 3)
    return out.reshape(B, S, D)
